# baseline (speedup 1.0000x reference)
import functools

import jax
import jax.numpy as jnp
from jax import lax
from jax.experimental import pallas as pl
from jax.experimental.pallas import tpu as pltpu
from jax.experimental.pallas import tpu_sc as plsc

_N = 10000
_E = 320000
_D = 128

_NC = 2
_NS = 16
_HALF = _N // _NC
_EPT = _E // _NS
_CH = 125
_NCHUNK = _EPT // _CH
_NPC = 5120
_TRASH = 5100
_RPT = _NPC // _NS
_ZR = 80


def _agg_body(h_hbm, src_hbm, dst_hbm, zero_hbm, out_hbm,
              src_v, dst_v, rows_v, acc_sh,
              gsem0, gsem1, ssem0, ssem1):
    c = lax.axis_index("c")
    s = lax.axis_index("s")

    pltpu.sync_copy(src_hbm.at[s], src_v)
    pltpu.sync_copy(dst_hbm.at[c, s], dst_v)

    for k in range(_RPT // _ZR):
        pltpu.sync_copy(zero_hbm, acc_sh.at[pl.ds(s * _RPT + k * _ZR, _ZR)])
    plsc.subcore_barrier()

    gsems = (gsem0, gsem1)
    ssems = (ssem0, ssem1)

    def _start_gather(j, b):
        pltpu.async_copy(h_hbm.at[src_v.at[j]], rows_v.at[b], gsems[b])

    def _wait_gather(j, b):
        pltpu.make_async_copy(h_hbm.at[src_v.at[j]], rows_v.at[b],
                              gsems[b]).wait()

    def _start_scatter(j, b):
        pltpu.async_copy(rows_v.at[b], acc_sh.at[dst_v.at[j]], ssems[b],
                         add=True)

    def _wait_scatter(j, b):
        pltpu.make_async_copy(rows_v.at[b], acc_sh.at[dst_v.at[j]],
                              ssems[b]).wait()

    _start_gather(0, 0)

    def body(i, carry):
        for b in range(2):
            j = 2 * i + b
            _wait_gather(j, b)
            _start_scatter(j, b)

            @pl.when(j >= 1)
            def _():
                _wait_scatter(j - 1, 1 - b)

            @pl.when(j + 1 < _NCHUNK)
            def _():
                _start_gather(j + 1, 1 - b)
        return carry

    lax.fori_loop(0, _NCHUNK // 2, body, 0)
    _wait_scatter(_NCHUNK - 1, 1)

    plsc.subcore_barrier()
    pltpu.sync_copy(acc_sh.at[pl.ds(s * _RPT, _RPT)],
                    out_hbm.at[c, pl.ds(s * _RPT, _RPT)])


_agg = pl.kernel(
    _agg_body,
    out_type=jax.ShapeDtypeStruct((_NC, _NPC, _D), jnp.float32),
    mesh=plsc.VectorSubcoreMesh(core_axis_name="c", subcore_axis_name="s"),
    scratch_types=[
        pltpu.VMEM((_NCHUNK, _CH), jnp.int32),
        pltpu.VMEM((_NCHUNK, _CH), jnp.int32),
        pltpu.VMEM((2, _CH, _D), jnp.float32),
        pltpu.VMEM_SHARED((_NPC, _D), jnp.float32),
        pltpu.SemaphoreType.DMA,
        pltpu.SemaphoreType.DMA,
        pltpu.SemaphoreType.DMA,
        pltpu.SemaphoreType.DMA,
    ],
)


_BLK = 1000
_GRID = _N // _BLK
_BPC = _HALF // _BLK

_row_spec = pl.BlockSpec((_BLK, _D), lambda i: (i, 0))
_p_spec = pl.BlockSpec((1, _BLK, _D), lambda i: (i // _BPC, i % _BPC, 0))
_w_spec = pl.BlockSpec((_D, _D), lambda i: (0, 0))
_b_spec = pl.BlockSpec((1, _D), lambda i: (0, 0))
_out_struct = jax.ShapeDtypeStruct((_N, _D), jnp.float32)


def _mm_body(x_ref, w_ref, o_ref):
    o_ref[...] = jnp.dot(x_ref[...], w_ref[...],
                         preferred_element_type=jnp.float32)


def _relu_mm_body(p_ref, b_ref, w_ref, o_ref):
    z = jnp.maximum(p_ref[0] + b_ref[...], 0.0)
    o_ref[...] = jnp.dot(z, w_ref[...], preferred_element_type=jnp.float32)


def _sigmoid_body(p_ref, b_ref, o_ref):
    o_ref[...] = jax.nn.sigmoid(p_ref[0] + b_ref[...])


_mm = pl.pallas_call(
    _mm_body, grid=(_GRID,),
    in_specs=[_row_spec, _w_spec],
    out_specs=_row_spec, out_shape=_out_struct)

_relu_mm = pl.pallas_call(
    _relu_mm_body, grid=(_GRID,),
    in_specs=[_p_spec, _b_spec, _w_spec],
    out_specs=_row_spec, out_shape=_out_struct)

_sigmoid = pl.pallas_call(
    _sigmoid_body, grid=(_GRID,),
    in_specs=[_p_spec, _b_spec],
    out_specs=_row_spec, out_shape=_out_struct)


def kernel(x, edge_index, W1, b1, W2, b2, W3, b3):
    src = edge_index[0].reshape(_NS, _NCHUNK, _CH)
    dst = edge_index[1]
    dst_loc = jnp.stack([
        jnp.where(dst < _HALF, dst, _TRASH),
        jnp.where(dst >= _HALF, dst - _HALF, _TRASH),
    ]).reshape(_NC, _NS, _NCHUNK, _CH)
    zero = jnp.zeros((_ZR, _D), jnp.float32)

    h = _mm(x, W1)
    p = _agg(h, src, dst_loc, zero)
    h = _relu_mm(p, b1.reshape(1, _D), W2)
    p = _agg(h, src, dst_loc, zero)
    h = _relu_mm(p, b2.reshape(1, _D), W3)
    p = _agg(h, src, dst_loc, zero)
    return _sigmoid(p, b3.reshape(1, _D))


# speedup vs baseline: 1.0360x; 1.0360x over previous
import functools

import jax
import jax.numpy as jnp
from jax import lax
from jax.experimental import pallas as pl
from jax.experimental.pallas import tpu as pltpu
from jax.experimental.pallas import tpu_sc as plsc

_N = 10000
_E = 320000
_D = 128

_NC = 2
_NS = 16
_HALF = _N // _NC
_EPT = _E // _NS
_CH = 125
_NCHUNK = _EPT // _CH
_NPC = 6400
_NTRASH = _NPC - _HALF
_ZPT = _NPC // _NS
_OPC = 5120
_RPT = _OPC // _NS
_ZR = 80


def _agg_body(h_hbm, src_hbm, dst_hbm, zero_hbm, out_hbm,
              src_v, dst_v, rows_v, acc_sh,
              gsem0, gsem1, ssem0, ssem1):
    c = lax.axis_index("c")
    s = lax.axis_index("s")

    pltpu.sync_copy(src_hbm.at[s], src_v)
    pltpu.sync_copy(dst_hbm.at[c, s], dst_v)

    for k in range(_ZPT // _ZR):
        pltpu.sync_copy(zero_hbm, acc_sh.at[pl.ds(s * _ZPT + k * _ZR, _ZR)])
    plsc.subcore_barrier()

    gsems = (gsem0, gsem1)
    ssems = (ssem0, ssem1)

    def _start_gather(j, b):
        pltpu.async_copy(h_hbm.at[src_v.at[j]], rows_v.at[b], gsems[b])

    def _wait_gather(j, b):
        pltpu.make_async_copy(h_hbm.at[src_v.at[j]], rows_v.at[b],
                              gsems[b]).wait()

    def _start_scatter(j, b):
        pltpu.async_copy(rows_v.at[b], acc_sh.at[dst_v.at[j]], ssems[b],
                         add=True)

    def _wait_scatter(j, b):
        pltpu.make_async_copy(rows_v.at[b], acc_sh.at[dst_v.at[j]],
                              ssems[b]).wait()

    _start_gather(0, 0)

    def body(i, carry):
        for b in range(2):
            j = 2 * i + b
            _wait_gather(j, b)
            _start_scatter(j, b)

            @pl.when(j >= 1)
            def _():
                _wait_scatter(j - 1, 1 - b)

            @pl.when(j + 1 < _NCHUNK)
            def _():
                _start_gather(j + 1, 1 - b)
        return carry

    lax.fori_loop(0, _NCHUNK // 2, body, 0)
    _wait_scatter(_NCHUNK - 1, 1)

    plsc.subcore_barrier()
    pltpu.sync_copy(acc_sh.at[pl.ds(s * _RPT, _RPT)],
                    out_hbm.at[c, pl.ds(s * _RPT, _RPT)])


_agg = pl.kernel(
    _agg_body,
    out_type=jax.ShapeDtypeStruct((_NC, _OPC, _D), jnp.float32),
    mesh=plsc.VectorSubcoreMesh(core_axis_name="c", subcore_axis_name="s"),
    scratch_types=[
        pltpu.VMEM((_NCHUNK, _CH), jnp.int32),
        pltpu.VMEM((_NCHUNK, _CH), jnp.int32),
        pltpu.VMEM((2, _CH, _D), jnp.float32),
        pltpu.VMEM_SHARED((_NPC, _D), jnp.float32),
        pltpu.SemaphoreType.DMA,
        pltpu.SemaphoreType.DMA,
        pltpu.SemaphoreType.DMA,
        pltpu.SemaphoreType.DMA,
    ],
)


_BLK = 1000
_GRID = _N // _BLK
_BPC = _HALF // _BLK

_row_spec = pl.BlockSpec((_BLK, _D), lambda i: (i, 0))
_p_spec = pl.BlockSpec((1, _BLK, _D), lambda i: (i // _BPC, i % _BPC, 0))
_w_spec = pl.BlockSpec((_D, _D), lambda i: (0, 0))
_b_spec = pl.BlockSpec((1, _D), lambda i: (0, 0))
_out_struct = jax.ShapeDtypeStruct((_N, _D), jnp.float32)


def _mm_body(x_ref, w_ref, o_ref):
    o_ref[...] = jnp.dot(x_ref[...], w_ref[...],
                         preferred_element_type=jnp.float32)


def _relu_mm_body(p_ref, b_ref, w_ref, o_ref):
    z = jnp.maximum(p_ref[0] + b_ref[...], 0.0)
    o_ref[...] = jnp.dot(z, w_ref[...], preferred_element_type=jnp.float32)


def _sigmoid_body(p_ref, b_ref, o_ref):
    o_ref[...] = jax.nn.sigmoid(p_ref[0] + b_ref[...])


_mm = pl.pallas_call(
    _mm_body, grid=(_GRID,),
    in_specs=[_row_spec, _w_spec],
    out_specs=_row_spec, out_shape=_out_struct)

_relu_mm = pl.pallas_call(
    _relu_mm_body, grid=(_GRID,),
    in_specs=[_p_spec, _b_spec, _w_spec],
    out_specs=_row_spec, out_shape=_out_struct)

_sigmoid = pl.pallas_call(
    _sigmoid_body, grid=(_GRID,),
    in_specs=[_p_spec, _b_spec],
    out_specs=_row_spec, out_shape=_out_struct)


def kernel(x, edge_index, W1, b1, W2, b2, W3, b3):
    src = edge_index[0].reshape(_NS, _NCHUNK, _CH)
    dst = edge_index[1]
    trash = _HALF + (jnp.arange(_E, dtype=jnp.int32) % _NTRASH)
    dst_loc = jnp.stack([
        jnp.where(dst < _HALF, dst, trash),
        jnp.where(dst >= _HALF, dst - _HALF, trash),
    ]).reshape(_NC, _NS, _NCHUNK, _CH)
    zero = jnp.zeros((_ZR, _D), jnp.float32)

    h = _mm(x, W1)
    p = _agg(h, src, dst_loc, zero)
    h = _relu_mm(p, b1.reshape(1, _D), W2)
    p = _agg(h, src, dst_loc, zero)
    h = _relu_mm(p, b2.reshape(1, _D), W3)
    p = _agg(h, src, dst_loc, zero)
    return _sigmoid(p, b3.reshape(1, _D))
